# baseline (speedup 1.0000x reference)
import jax
import jax.numpy as jnp
from jax import lax
from jax.experimental import pallas as pl
from jax.experimental.pallas import tpu as pltpu
from jax.experimental.pallas import tpu_sc as plsc

NC = 2
NS = 16
S = 10000
HALF = S // 2
N = 320000
D = 128

C = 80
NTOT = N // C
NWIN = N // 16
TRASH = HALF
AR = HALF + 8
USML = HALF // NS
USBIG = HALF - 15 * USML


def _sc_body(u_hbm, v_hbm, b_hbm, out_hbm, vbuf_a, vbuf_b, vbuf_c, vbuf_d, vbuf_e, vbuf_f,
             iring, sbuf_v, acc, sem_in, sem_ix, sem_u, sem_sc):
  c = lax.axis_index("c")
  s = lax.axis_index("s")
  bufs = [vbuf_a, vbuf_b, vbuf_c, vbuf_d, vbuf_e, vbuf_f]

  @pl.when(s < NS - 1)
  def _():
    pltpu.async_copy(u_hbm.at[pl.ds(c * HALF + s * USML, USML)],
                     acc.at[pl.ds(s * USML, USML)], sem_u)
  @pl.when(s == NS - 1)
  def _():
    pltpu.async_copy(u_hbm.at[pl.ds(c * HALF + 15 * USML, USBIG)],
                     acc.at[pl.ds(15 * USML, USBIG)], sem_u)

  def bs_round(_, carry):
    lo, hi = carry
    done = lo >= hi
    wi = jnp.minimum((lo + hi) // 2, N // 8 - 1)
    p = 8 * wi
    wstart = jnp.minimum(p, N - 16)
    pltpu.sync_copy(b_hbm.at[pl.ds(wstart, 16)], sbuf_v)
    vec = sbuf_v[...]
    val = jnp.where(p == wstart, vec[0], vec[8])
    pred = val >= HALF
    return (jnp.where(done, lo, jnp.where(pred, lo, wi + 1)),
            jnp.where(done, hi, jnp.where(pred, wi, hi)))
  lo8, _ = lax.fori_loop(0, 16, bs_round, (jnp.int32(0), jnp.int32(N // 8)))
  rstar = 8 * lo8

  k_split = (rstar + (C - 1)) // C
  start = jnp.where(c == 0, 0, jnp.maximum(k_split - 1, 0))
  end = jnp.where(c == 0, k_split, NTOT)
  t_cnt = jnp.maximum((end - start - s + (NS - 1)) // NS, 0)
  base = start + s

  @pl.when(s < NS - 1)
  def _():
    pltpu.make_async_copy(u_hbm.at[pl.ds(0, USML)], acc.at[pl.ds(0, USML)],
                          sem_u).wait()
  @pl.when(s == NS - 1)
  def _():
    pltpu.make_async_copy(u_hbm.at[pl.ds(0, USBIG)], acc.at[pl.ds(0, USBIG)],
                          sem_u).wait()
  plsc.subcore_barrier()

  lo_vec = jnp.full((16,), 0, jnp.int32)
  hi_vec = jnp.full((16,), HALF, jnp.int32)
  trash16 = jnp.full((16,), TRASH, jnp.int32)

  def ingest(j, b):
    k = base + NS * j
    pltpu.async_copy(v_hbm.at[pl.ds(k * C, C)], bufs[b], sem_in)
    pltpu.async_copy(b_hbm.at[pl.ds(k * C, C)], iring.at[b], sem_ix)

  @pl.when(t_cnt > 0)
  def _():
    ingest(0, 0)
  @pl.when(t_cnt > 1)
  def _():
    ingest(1, 1)

  cbase = c * HALF

  def step(j, b):
    @pl.when(j + 2 < t_cnt)
    def _():
      ingest(j + 2, (b + 2) % 6)
    pltpu.make_async_copy(v_hbm.at[pl.ds(0, C)], bufs[b], sem_in).wait()
    pltpu.make_async_copy(b_hbm.at[pl.ds(0, C)], iring.at[b], sem_ix).wait()
    for q in range(C // 16):
      w = iring[b, pl.ds(q * 16, 16)] - cbase
      bad = (w < lo_vec) | (w >= hi_vec)
      iring[b, pl.ds(q * 16, 16)] = jnp.where(bad, trash16, w)
    pltpu.async_copy(bufs[b], acc.at[iring.at[b]], sem_sc, add=True)

  def body(j, _):
    @pl.when(j >= 4)
    def _():
      pltpu.make_async_copy(v_hbm.at[pl.ds(0, C)], vbuf_a, sem_sc).wait()
    for b in range(6):
      @pl.when(j % 6 == b)
      def _():
        step(j, b)
    return 0
  lax.fori_loop(0, t_cnt, body, 0)
  for r in range(4):
    @pl.when(t_cnt > r)
    def _():
      pltpu.make_async_copy(v_hbm.at[pl.ds(0, C)], vbuf_a, sem_sc).wait()
  plsc.subcore_barrier()

  @pl.when(s < NS - 1)
  def _():
    pltpu.sync_copy(acc.at[pl.ds(s * USML, USML)],
                    out_hbm.at[pl.ds(c * HALF + s * USML, USML)])
  @pl.when(s == NS - 1)
  def _():
    pltpu.sync_copy(acc.at[pl.ds(15 * USML, USBIG)],
                    out_hbm.at[pl.ds(c * HALF + 15 * USML, USBIG)])


_sc_kernel = pl.kernel(
    _sc_body,
    out_type=jax.ShapeDtypeStruct((S, D), jnp.float32),
    mesh=plsc.VectorSubcoreMesh(core_axis_name="c", subcore_axis_name="s"),
    scratch_types=[
        pltpu.VMEM((C, D), jnp.float32),
        pltpu.VMEM((C, D), jnp.float32),
        pltpu.VMEM((C, D), jnp.float32),
        pltpu.VMEM((C, D), jnp.float32),
        pltpu.VMEM((C, D), jnp.float32),
        pltpu.VMEM((C, D), jnp.float32),
        pltpu.VMEM((6, C), jnp.int32),
        pltpu.VMEM((16,), jnp.int32),
        pltpu.VMEM_SHARED((AR, D), jnp.float32),
        pltpu.SemaphoreType.DMA,
        pltpu.SemaphoreType.DMA,
        pltpu.SemaphoreType.DMA,
        pltpu.SemaphoreType.DMA,
    ],
)


@jax.jit
def kernel(u, v, batch):
  return _sc_kernel(u, v, batch)


# speedup vs baseline: 1.0146x; 1.0146x over previous
import jax
import jax.numpy as jnp
from jax import lax
from jax.experimental import pallas as pl
from jax.experimental.pallas import tpu as pltpu
from jax.experimental.pallas import tpu_sc as plsc

NC = 2
NS = 16
S = 10000
HALF = S // 2
N = 320000
D = 128

C = 80
NTOT = N // C
NWIN = N // 16
TRASH = HALF
AR = HALF + 8
USML = HALF // NS
USBIG = HALF - 15 * USML


def _sc_body(u_hbm, v_hbm, b_hbm, out_hbm, vbuf_a, vbuf_b, vbuf_c, vbuf_d,
             iring, sbuf_v, acc, sem_in, sem_ix, sem_u, sem_sc):
  c = lax.axis_index("c")
  s = lax.axis_index("s")
  bufs = [vbuf_a, vbuf_b, vbuf_c, vbuf_d]

  @pl.when(s < NS - 1)
  def _():
    pltpu.async_copy(u_hbm.at[pl.ds(c * HALF + s * USML, USML)],
                     acc.at[pl.ds(s * USML, USML)], sem_u)
  @pl.when(s == NS - 1)
  def _():
    pltpu.async_copy(u_hbm.at[pl.ds(c * HALF + 15 * USML, USBIG)],
                     acc.at[pl.ds(15 * USML, USBIG)], sem_u)

  def bs_round(_, carry):
    lo, hi = carry
    done = lo >= hi
    wi = jnp.minimum((lo + hi) // 2, N // 8 - 1)
    p = 8 * wi
    wstart = jnp.minimum(p, N - 16)
    pltpu.sync_copy(b_hbm.at[pl.ds(wstart, 16)], sbuf_v)
    vec = sbuf_v[...]
    val = jnp.where(p == wstart, vec[0], vec[8])
    pred = val >= HALF
    return (jnp.where(done, lo, jnp.where(pred, lo, wi + 1)),
            jnp.where(done, hi, jnp.where(pred, wi, hi)))
  lo8, _ = lax.fori_loop(0, 16, bs_round, (jnp.int32(0), jnp.int32(N // 8)))
  rstar = 8 * lo8

  k_split = (rstar + (C - 1)) // C
  start = jnp.where(c == 0, 0, jnp.maximum(k_split - 1, 0))
  end = jnp.where(c == 0, k_split, NTOT)
  t_cnt = jnp.maximum((end - start - s + (NS - 1)) // NS, 0)
  base = start + s

  @pl.when(s < NS - 1)
  def _():
    pltpu.make_async_copy(u_hbm.at[pl.ds(0, USML)], acc.at[pl.ds(0, USML)],
                          sem_u).wait()
  @pl.when(s == NS - 1)
  def _():
    pltpu.make_async_copy(u_hbm.at[pl.ds(0, USBIG)], acc.at[pl.ds(0, USBIG)],
                          sem_u).wait()
  plsc.subcore_barrier()

  lo_vec = jnp.full((16,), 0, jnp.int32)
  hi_vec = jnp.full((16,), HALF, jnp.int32)
  trash16 = jnp.full((16,), TRASH, jnp.int32)

  def ingest(j, b):
    k = base + NS * j
    pltpu.async_copy(v_hbm.at[pl.ds(k * C, C)], bufs[b], sem_in)
    pltpu.async_copy(b_hbm.at[pl.ds(k * C, C)], iring.at[b], sem_ix)

  @pl.when(t_cnt > 0)
  def _():
    ingest(0, 0)
  @pl.when(t_cnt > 1)
  def _():
    ingest(1, 1)

  cbase = c * HALF

  def step(j, b):
    @pl.when(j + 2 < t_cnt)
    def _():
      ingest(j + 2, (b + 2) % 4)
    pltpu.make_async_copy(v_hbm.at[pl.ds(0, C)], bufs[b], sem_in).wait()
    pltpu.make_async_copy(b_hbm.at[pl.ds(0, C)], iring.at[b], sem_ix).wait()
    for q in range(C // 16):
      w = iring[b, pl.ds(q * 16, 16)] - cbase
      bad = (w < lo_vec) | (w >= hi_vec)
      iring[b, pl.ds(q * 16, 16)] = jnp.where(bad, trash16, w)
    pltpu.async_copy(bufs[b], acc.at[iring.at[b]], sem_sc, add=True)

  def body(j, _):
    @pl.when(j >= 2)
    def _():
      pltpu.make_async_copy(v_hbm.at[pl.ds(0, C)], vbuf_a, sem_sc).wait()
    for b in range(4):
      @pl.when(j % 4 == b)
      def _():
        step(j, b)
    return 0
  lax.fori_loop(0, t_cnt, body, 0)
  @pl.when(t_cnt > 0)
  def _():
    pltpu.make_async_copy(v_hbm.at[pl.ds(0, C)], vbuf_a, sem_sc).wait()
  @pl.when(t_cnt > 1)
  def _():
    pltpu.make_async_copy(v_hbm.at[pl.ds(0, C)], vbuf_a, sem_sc).wait()
  plsc.subcore_barrier()

  @pl.when(s < NS - 1)
  def _():
    pltpu.sync_copy(acc.at[pl.ds(s * USML, USML)],
                    out_hbm.at[pl.ds(c * HALF + s * USML, USML)])
  @pl.when(s == NS - 1)
  def _():
    pltpu.sync_copy(acc.at[pl.ds(15 * USML, USBIG)],
                    out_hbm.at[pl.ds(c * HALF + 15 * USML, USBIG)])


_sc_kernel = pl.kernel(
    _sc_body,
    out_type=jax.ShapeDtypeStruct((S, D), jnp.float32),
    mesh=plsc.VectorSubcoreMesh(core_axis_name="c", subcore_axis_name="s"),
    scratch_types=[
        pltpu.VMEM((C, D), jnp.float32),
        pltpu.VMEM((C, D), jnp.float32),
        pltpu.VMEM((C, D), jnp.float32),
        pltpu.VMEM((C, D), jnp.float32),
        pltpu.VMEM((4, C), jnp.int32),
        pltpu.VMEM((16,), jnp.int32),
        pltpu.VMEM_SHARED((AR, D), jnp.float32),
        pltpu.SemaphoreType.DMA,
        pltpu.SemaphoreType.DMA,
        pltpu.SemaphoreType.DMA,
        pltpu.SemaphoreType.DMA,
    ],
)


@jax.jit
def kernel(u, v, batch):
  return _sc_kernel(u, v, batch)
